# Initial kernel scaffold; baseline (speedup 1.0000x reference)
import jax
import jax.numpy as jnp
from jax.experimental import pallas as pl
from jax.experimental.pallas import tpu as pltpu

NUM_SEGMENTS = 512
BV = 2000


def _agg_kernel(h_ref, batch_ref, w_ref, b_ref, out_ref, m_ref, den_ref, acc_ref):
    i = pl.program_id(0)
    nb = pl.num_programs(0)

    @pl.when(i == 0)
    def _init():
        m_ref[...] = jnp.full_like(m_ref, -jnp.inf)
        den_ref[...] = jnp.zeros_like(den_ref)
        acc_ref[...] = jnp.zeros_like(acc_ref)

    h = h_ref[...]
    batch = batch_ref[0]
    scores = jnp.dot(h, w_ref[...], preferred_element_type=jnp.float32)
    scores = scores + b_ref[0, 0]

    seg_ids = jax.lax.broadcasted_iota(jnp.int32, (NUM_SEGMENTS, BV), 0)
    maskb = seg_ids == batch
    mask = maskb.astype(jnp.float32)

    scores_t = scores.reshape(1, BV)
    blk_max = jnp.max(jnp.where(maskb, scores_t, -jnp.inf), axis=1, keepdims=True)

    m_old = m_ref[...]
    m_new = jnp.maximum(m_old, blk_max)
    scale = jnp.where(m_old == -jnp.inf, 0.0, jnp.exp(m_old - m_new))

    row_m = jnp.sum(mask * m_new, axis=0).reshape(BV, 1)
    p = jnp.exp(scores - row_m)

    den_ref[...] = den_ref[...] * scale + jnp.sum(
        mask * p.reshape(1, BV), axis=1, keepdims=True
    )
    weighted = p * h
    acc_ref[...] = acc_ref[...] * scale + jnp.dot(
        mask, weighted, preferred_element_type=jnp.float32
    )
    m_ref[...] = m_new

    @pl.when(i == nb - 1)
    def _fini():
        den = den_ref[...]
        out_ref[...] = jnp.where(den > 0.0, acc_ref[...] / den, 0.0)


@jax.jit
def kernel(H, batch, W, b):
    V, D = H.shape
    nb = V // BV
    batch_r = batch.astype(jnp.int32).reshape(nb, 1, BV)
    b_r = b.reshape(1, 1).astype(jnp.float32)

    out = pl.pallas_call(
        _agg_kernel,
        grid=(nb,),
        in_specs=[
            pl.BlockSpec((BV, D), lambda i: (i, 0)),
            pl.BlockSpec((1, 1, BV), lambda i: (i, 0, 0)),
            pl.BlockSpec((D, 1), lambda i: (0, 0)),
            pl.BlockSpec((1, 1), lambda i: (0, 0)),
        ],
        out_specs=pl.BlockSpec((NUM_SEGMENTS, D), lambda i: (0, 0)),
        out_shape=jax.ShapeDtypeStruct((NUM_SEGMENTS, D), jnp.float32),
        scratch_shapes=[
            pltpu.VMEM((NUM_SEGMENTS, 1), jnp.float32),
            pltpu.VMEM((NUM_SEGMENTS, 1), jnp.float32),
            pltpu.VMEM((NUM_SEGMENTS, D), jnp.float32),
        ],
    )(H, batch_r, W, b_r)
    return out


# speedup vs baseline: 7.3570x; 7.3570x over previous
import jax
import jax.numpy as jnp
from jax.experimental import pallas as pl
from jax.experimental.pallas import tpu as pltpu

NUM_SEGMENTS = 512
BV = 2048
_NEG_BIG = -1e30


def _agg_kernel(h_ref, batch_ref, w_ref, b_ref, out_ref, m_ref, den_ref, acc_ref):
    i = pl.program_id(0)
    nb = pl.num_programs(0)

    @pl.when(i == 0)
    def _init():
        m_ref[...] = jnp.full_like(m_ref, _NEG_BIG)
        den_ref[...] = jnp.zeros_like(den_ref)
        acc_ref[...] = jnp.zeros_like(acc_ref)

    h = h_ref[...]
    batch = batch_ref[0]
    scores = jnp.dot(h, w_ref[...], preferred_element_type=jnp.float32)
    scores = scores + b_ref[0, 0]

    seg_ids = jax.lax.broadcasted_iota(jnp.int32, (NUM_SEGMENTS, BV), 0)
    maskb = seg_ids == batch
    mask = maskb.astype(jnp.float32)

    scores_t = scores.reshape(1, BV)
    blk_max = jnp.max(jnp.where(maskb, scores_t, _NEG_BIG), axis=1, keepdims=True)

    m_old = m_ref[...]
    m_new = jnp.maximum(m_old, blk_max)
    scale = jnp.exp(m_old - m_new)

    row_m = jnp.sum(mask * m_new, axis=0).reshape(BV, 1)
    p = jnp.exp(scores - row_m)

    den_ref[...] = den_ref[...] * scale + jnp.sum(
        mask * p.reshape(1, BV), axis=1, keepdims=True
    )
    weighted = p * h
    acc_ref[...] = acc_ref[...] * scale + jnp.dot(
        mask, weighted, preferred_element_type=jnp.float32
    )
    m_ref[...] = m_new

    @pl.when(i == nb - 1)
    def _fini():
        den = den_ref[...]
        out_ref[...] = jnp.where(den > 0.0, acc_ref[...] / den, 0.0)


@jax.jit
def kernel(H, batch, W, b):
    V, D = H.shape
    nb = (V + BV - 1) // BV
    vpad = nb * BV - V
    if vpad:
        H = jnp.concatenate([H, jnp.zeros((vpad, D), H.dtype)], axis=0)
        batch = jnp.concatenate(
            [batch.astype(jnp.int32), jnp.full((vpad,), NUM_SEGMENTS, jnp.int32)]
        )
    batch_r = batch.astype(jnp.int32).reshape(nb, 1, BV)
    b_r = b.reshape(1, 1).astype(jnp.float32)

    out = pl.pallas_call(
        _agg_kernel,
        grid=(nb,),
        in_specs=[
            pl.BlockSpec((BV, D), lambda i: (i, 0)),
            pl.BlockSpec((1, 1, BV), lambda i: (i, 0, 0)),
            pl.BlockSpec((D, 1), lambda i: (0, 0)),
            pl.BlockSpec((1, 1), lambda i: (0, 0)),
        ],
        out_specs=pl.BlockSpec((NUM_SEGMENTS, D), lambda i: (0, 0)),
        out_shape=jax.ShapeDtypeStruct((NUM_SEGMENTS, D), jnp.float32),
        scratch_shapes=[
            pltpu.VMEM((NUM_SEGMENTS, 1), jnp.float32),
            pltpu.VMEM((NUM_SEGMENTS, 1), jnp.float32),
            pltpu.VMEM((NUM_SEGMENTS, D), jnp.float32),
        ],
    )(H, batch_r, W, b_r)
    return out


# speedup vs baseline: 7.3617x; 1.0006x over previous
import jax
import jax.numpy as jnp
from jax.experimental import pallas as pl
from jax.experimental.pallas import tpu as pltpu

NUM_SEGMENTS = 512
BV = 2048
_NEG_BIG = -1e30


def _agg_kernel(h_ref, batch_ref, w_ref, b_ref, out_ref, m_ref, den_ref, acc_ref):
    i = pl.program_id(0)
    nb = pl.num_programs(0)

    @pl.when(i == 0)
    def _init():
        m_ref[...] = jnp.full_like(m_ref, _NEG_BIG)
        den_ref[...] = jnp.zeros_like(den_ref)
        acc_ref[...] = jnp.zeros_like(acc_ref)

    h = h_ref[...]
    batch = batch_ref[0]
    scores = jnp.dot(h, w_ref[...], preferred_element_type=jnp.float32)
    scores = scores + b_ref[0, 0]

    seg_ids = jax.lax.broadcasted_iota(jnp.int32, (NUM_SEGMENTS, BV), 0)
    maskb = seg_ids == batch
    mask = maskb.astype(jnp.float32)
    mask_bf = maskb.astype(jnp.bfloat16)

    scores_t = scores.reshape(1, BV)
    blk_max = jnp.max(jnp.where(maskb, scores_t, _NEG_BIG), axis=1, keepdims=True)

    m_old = m_ref[...]
    m_new = jnp.maximum(m_old, blk_max)
    scale = jnp.exp(m_old - m_new)

    row_m = jnp.sum(mask * m_new, axis=0).reshape(BV, 1)
    p = jnp.exp(scores - row_m)

    den_ref[...] = den_ref[...] * scale + jnp.sum(
        mask * p.reshape(1, BV), axis=1, keepdims=True
    )
    weighted = (p * h).astype(jnp.bfloat16)
    acc_ref[...] = acc_ref[...] * scale + jnp.dot(
        mask_bf, weighted, preferred_element_type=jnp.float32
    )
    m_ref[...] = m_new

    @pl.when(i == nb - 1)
    def _fini():
        den = den_ref[...]
        out_ref[...] = jnp.where(den > 0.0, acc_ref[...] / den, 0.0)


@jax.jit
def kernel(H, batch, W, b):
    V, D = H.shape
    nb = (V + BV - 1) // BV
    vpad = nb * BV - V
    if vpad:
        H = jnp.concatenate([H, jnp.zeros((vpad, D), H.dtype)], axis=0)
        batch = jnp.concatenate(
            [batch.astype(jnp.int32), jnp.full((vpad,), NUM_SEGMENTS, jnp.int32)]
        )
    batch_r = batch.astype(jnp.int32).reshape(nb, 1, BV)
    b_r = b.reshape(1, 1).astype(jnp.float32)

    out = pl.pallas_call(
        _agg_kernel,
        grid=(nb,),
        in_specs=[
            pl.BlockSpec((BV, D), lambda i: (i, 0)),
            pl.BlockSpec((1, 1, BV), lambda i: (i, 0, 0)),
            pl.BlockSpec((D, 1), lambda i: (0, 0)),
            pl.BlockSpec((1, 1), lambda i: (0, 0)),
        ],
        out_specs=pl.BlockSpec((NUM_SEGMENTS, D), lambda i: (0, 0)),
        out_shape=jax.ShapeDtypeStruct((NUM_SEGMENTS, D), jnp.float32),
        scratch_shapes=[
            pltpu.VMEM((NUM_SEGMENTS, 1), jnp.float32),
            pltpu.VMEM((NUM_SEGMENTS, 1), jnp.float32),
            pltpu.VMEM((NUM_SEGMENTS, D), jnp.float32),
        ],
    )(H, batch_r, W, b_r)
    return out
